# baseline (speedup 1.0000x reference)
import functools

import jax
import jax.numpy as jnp
from jax import lax
from jax.experimental import pallas as pl
from jax.experimental.pallas import tpu as pltpu
from jax.experimental.pallas import tpu_sc as plsc

B = 4096
L = 200
EMB = 64
NCLS = 100

NC, NS = 2, 16
NW = NC * NS
RPW = B // NW
C0 = 128
C1 = L - C0
NQ = EMB // 16


def _sc_pool_sum(x, emb_table):
    mesh = plsc.VectorSubcoreMesh(core_axis_name="c", subcore_axis_name="s")

    @functools.partial(
        pl.kernel,
        out_type=jax.ShapeDtypeStruct((B, EMB), jnp.float32),
        mesh=mesh,
        compiler_params=pltpu.CompilerParams(use_tc_tiling_on_sc=False),
        scratch_types=[
            pltpu.VMEM((RPW, L), jnp.int32),
            pltpu.VMEM((L, EMB), jnp.float32),
            pltpu.VMEM((L, EMB), jnp.float32),
            pltpu.VMEM((RPW, EMB), jnp.float32),
            pltpu.SemaphoreType.DMA,
            pltpu.SemaphoreType.DMA,
        ],
    )
    def pool_kernel(x_hbm, tab_hbm, out_hbm, idx_v, rows0, rows1, acc_v, sem0, sem1):
        wid = lax.axis_index("s") * NC + lax.axis_index("c")
        base = wid * RPW
        pltpu.sync_copy(x_hbm.at[pl.ds(base, RPW), :], idx_v)

        def issue(r, rows_v, sem):
            pltpu.async_copy(
                tab_hbm.at[idx_v.at[r, pl.ds(0, C0)]], rows_v.at[pl.ds(0, C0), :], sem)
            pltpu.async_copy(
                tab_hbm.at[idx_v.at[r, pl.ds(C0, C1)]], rows_v.at[pl.ds(C0, C1), :], sem)

        def drain(rows_v, sem):
            pltpu.make_async_copy(tab_hbm.at[pl.ds(0, L), :], rows_v, sem).wait()

        def accum(r, rows_v):
            zero = jnp.zeros((16,), jnp.float32)

            def body(j, accs):
                return tuple(a + rows_v[j, pl.ds(16 * q, 16)]
                             for q, a in enumerate(accs))

            accs = lax.fori_loop(0, L, body, (zero,) * NQ)
            for q in range(NQ):
                acc_v[r, pl.ds(16 * q, 16)] = accs[q]

        issue(0, rows0, sem0)

        def outer(t, carry):
            r = 2 * t
            issue(r + 1, rows1, sem1)
            drain(rows0, sem0)
            accum(r, rows0)

            @pl.when(r + 2 < RPW)
            def _():
                issue(r + 2, rows0, sem0)

            drain(rows1, sem1)
            accum(r + 1, rows1)
            return carry

        lax.fori_loop(0, RPW // 2, outer, 0)
        pltpu.sync_copy(acc_v, out_hbm.at[pl.ds(base, RPW), :])

    return pool_kernel(x, emb_table)


def _tc_head(pooled_sum, Wt, b2):
    def head_kernel(p_ref, w_ref, b_ref, o_ref):
        o_ref[...] = (
            jnp.dot(p_ref[...], w_ref[...], preferred_element_type=jnp.float32)
            * (1.0 / L)
            + b_ref[...]
        )

    return pl.pallas_call(
        head_kernel,
        out_shape=jax.ShapeDtypeStruct((B, NCLS), jnp.float32),
    )(pooled_sum, Wt, b2)


def kernel(x, emb_table, W, b):
    x = x.astype(jnp.int32)
    pooled_sum = _sc_pool_sum(x, emb_table)
    return _tc_head(pooled_sum, W.T, b.reshape(1, NCLS))
